# baseline (speedup 1.0000x reference)
import functools

import jax
import jax.numpy as jnp
from jax import lax
from jax.experimental import pallas as pl
from jax.experimental.pallas import tpu as pltpu
from jax.experimental.pallas import tpu_sc as plsc

_NC = plsc.get_sparse_core_info().num_cores


def _make_sc_copy(T, D, dtype):
    rows = T // _NC
    mesh = plsc.ScalarSubcoreMesh(axis_name="c", num_cores=_NC)

    @functools.partial(
        pl.kernel,
        mesh=mesh,
        out_type=jax.ShapeDtypeStruct((T, D), dtype),
        scratch_types=[pltpu.SemaphoreType.DMA],
    )
    def _sc_copy(pe_hbm, out_hbm, sem):
        cid = lax.axis_index("c")
        base = cid * rows
        pltpu.async_copy(
            pe_hbm.at[pl.ds(base, rows), :],
            out_hbm.at[pl.ds(base, rows), :],
            sem,
        ).wait()

    return _sc_copy


def kernel(idx, pe):
    T = idx.shape[1]
    out = _make_sc_copy(T, pe.shape[1], pe.dtype)(pe)
    return out[None, :, :]


# speedup vs baseline: 17.6422x; 17.6422x over previous
import jax
import jax.numpy as jnp
from jax.experimental import pallas as pl
from jax.experimental.pallas import tpu as pltpu

_CHUNKS = 4


def _copy_body(pe_hbm, o_hbm, scratch, in_sems, out_sems):
    T = scratch.shape[0]
    rows = T // _CHUNKS
    for k in range(_CHUNKS):
        sl = pl.ds(k * rows, rows)
        pltpu.make_async_copy(
            pe_hbm.at[sl, :], scratch.at[sl, :], in_sems.at[k]
        ).start()
    for k in range(_CHUNKS):
        sl = pl.ds(k * rows, rows)
        pltpu.make_async_copy(
            pe_hbm.at[sl, :], scratch.at[sl, :], in_sems.at[k]
        ).wait()
        pltpu.make_async_copy(
            scratch.at[sl, :], o_hbm.at[0, sl, :], out_sems.at[k]
        ).start()
    for k in range(_CHUNKS):
        sl = pl.ds(k * rows, rows)
        pltpu.make_async_copy(
            scratch.at[sl, :], o_hbm.at[0, sl, :], out_sems.at[k]
        ).wait()


def kernel(idx, pe):
    T = idx.shape[1]
    D = pe.shape[1]
    return pl.pallas_call(
        _copy_body,
        out_shape=jax.ShapeDtypeStruct((1, T, D), pe.dtype),
        in_specs=[pl.BlockSpec(memory_space=pl.ANY)],
        out_specs=pl.BlockSpec(memory_space=pl.ANY),
        scratch_shapes=[
            pltpu.VMEM((T, D), pe.dtype),
            pltpu.SemaphoreType.DMA((_CHUNKS,)),
            pltpu.SemaphoreType.DMA((_CHUNKS,)),
        ],
    )(pe)
